# baseline (device time: 790262 ns/iter reference)
import jax
import jax.numpy as jnp
from jax import lax
from jax.experimental import pallas as pl
from jax.experimental.pallas import tpu as pltpu

N_DEV = 32
T = 256
D_SH = 4096
T_HALF = T // 2
N_SUB = 2
T_QTR = T_HALF // N_SUB
N_HOP = N_DEV - 1
LOG2 = 5


def kernel(x, W):
    logits = jnp.dot(
        x.astype(jnp.bfloat16),
        W.astype(jnp.bfloat16),
        preferred_element_type=jnp.float32,
    ).astype(jnp.bfloat16)

    def body(l_ref, out_ref, raw_ref, copy_sem,
             rsend_sems, rrecv_sems, lsend_sems, lrecv_sems,
             stats_send, stats_recv, ssend_sems, srecv_sems,
             tin_ref, tout_ref, tsem):
        my = lax.axis_index("i")
        left = lax.rem(my + N_DEV - 1, N_DEV)
        right = lax.rem(my + 1, N_DEV)

        partners = [my ^ (1 << k) for k in range(1, LOG2)]
        barrier = pltpu.get_barrier_semaphore()
        for nbr in [left, right] + partners:
            pl.semaphore_signal(
                barrier, inc=1,
                device_id=(nbr,), device_id_type=pl.DeviceIdType.MESH,
            )
        pl.semaphore_wait(barrier, 2 + len(partners))

        def origin_r(h):
            return lax.rem(my + (N_DEV - h), N_DEV)

        def origin_l(h):
            return lax.rem(my + h, N_DEV)

        def mk_r(h, j):
            o = origin_r(h)
            sub = raw_ref.at[o, pl.ds(j * T_QTR, T_QTR), :]
            return pltpu.make_async_remote_copy(
                src_ref=sub, dst_ref=sub,
                send_sem=rsend_sems.at[h * N_SUB + j],
                recv_sem=rrecv_sems.at[h * N_SUB + j],
                device_id=(right,),
                device_id_type=pl.DeviceIdType.MESH,
            )

        def mk_l(h, j):
            o = origin_l(h)
            sub = raw_ref.at[o, pl.ds(T_HALF + j * T_QTR, T_QTR), :]
            return pltpu.make_async_remote_copy(
                src_ref=sub, dst_ref=sub,
                send_sem=lsend_sems.at[h * N_SUB + j],
                recv_sem=lrecv_sems.at[h * N_SUB + j],
                device_id=(left,),
                device_id_type=pl.DeviceIdType.MESH,
            )

        cp = pltpu.make_async_copy(l_ref, raw_ref.at[my], copy_sem)
        cp.start()
        cp.wait()
        rdescs, ldescs = {}, {}
        for j in range(N_SUB):
            rdescs[(0, j)] = mk_r(0, j)
            rdescs[(0, j)].start()
            ldescs[(0, j)] = mk_l(0, j)
            ldescs[(0, j)].start()

        s = jnp.sum(jnp.exp(l_ref[:, :].astype(jnp.float32)),
                    axis=1, keepdims=True)
        for k in range(LOG2):
            stats_send[k, :, :] = s
            p = my ^ (1 << k)
            d = pltpu.make_async_remote_copy(
                src_ref=stats_send.at[k],
                dst_ref=stats_recv.at[k],
                send_sem=ssend_sems.at[k],
                recv_sem=srecv_sems.at[k],
                device_id=(p,),
                device_id_type=pl.DeviceIdType.MESH,
            )
            d.start()
            d.wait()
            s = s + stats_recv[k, :, :]
        inv_bf = (1.0 / s).astype(jnp.bfloat16)
        inv_top = inv_bf[:T_HALF]
        inv_bot = inv_bf[T_HALF:]

        def transform_half(o, row0, inv_half):
            ci = pltpu.make_async_copy(
                raw_ref.at[o, pl.ds(row0, T_HALF), :], tin_ref, tsem)
            ci.start()
            ci.wait()
            tout_ref[:, :] = jnp.exp(tin_ref[:, :]) * inv_half
            co = pltpu.make_async_copy(
                tout_ref,
                out_ref.at[pl.ds(row0, T_HALF), pl.ds(o * D_SH, D_SH)],
                tsem)
            co.start()
            co.wait()

        transform_half(my, 0, inv_top)
        transform_half(my, T_HALF, inv_bot)

        for h in range(1, N_HOP):
            for j in range(N_SUB):
                rdescs[(h - 1, j)].wait_recv()
                rdescs[(h, j)] = mk_r(h, j)
                rdescs[(h, j)].start()
            for j in range(N_SUB):
                ldescs[(h - 1, j)].wait_recv()
                ldescs[(h, j)] = mk_l(h, j)
                ldescs[(h, j)].start()
            transform_half(origin_r(h), 0, inv_top)
            transform_half(origin_l(h), T_HALF, inv_bot)

        for j in range(N_SUB):
            rdescs[(N_HOP - 1, j)].wait_recv()
            ldescs[(N_HOP - 1, j)].wait_recv()
        transform_half(origin_r(N_HOP), 0, inv_top)
        transform_half(origin_l(N_HOP), T_HALF, inv_bot)

        for h in range(N_HOP):
            for j in range(N_SUB):
                rdescs[(h, j)].wait_send()
                ldescs[(h, j)].wait_send()

    out, _raw = pl.pallas_call(
        body,
        out_shape=[
            jax.ShapeDtypeStruct((T, N_DEV * D_SH), jnp.bfloat16),
            jax.ShapeDtypeStruct((N_DEV, T, D_SH), jnp.bfloat16),
        ],
        in_specs=[pl.BlockSpec(memory_space=pltpu.VMEM)],
        out_specs=[
            pl.BlockSpec(memory_space=pl.ANY),
            pl.BlockSpec(memory_space=pl.ANY),
        ],
        scratch_shapes=[
            pltpu.SemaphoreType.DMA,
            pltpu.SemaphoreType.DMA((N_HOP * N_SUB,)),
            pltpu.SemaphoreType.DMA((N_HOP * N_SUB,)),
            pltpu.SemaphoreType.DMA((N_HOP * N_SUB,)),
            pltpu.SemaphoreType.DMA((N_HOP * N_SUB,)),
            pltpu.VMEM((LOG2, T, 1), jnp.float32),
            pltpu.VMEM((LOG2, T, 1), jnp.float32),
            pltpu.SemaphoreType.DMA((LOG2,)),
            pltpu.SemaphoreType.DMA((LOG2,)),
            pltpu.VMEM((T_HALF, D_SH), jnp.bfloat16),
            pltpu.VMEM((T_HALF, D_SH), jnp.bfloat16),
            pltpu.SemaphoreType.DMA,
        ],
        compiler_params=pltpu.CompilerParams(collective_id=0),
    )(logits)
    return out
